# baseline (speedup 1.0000x reference)
import functools

import jax
import jax.numpy as jnp
from jax import lax
from jax.experimental import pallas as pl
from jax.experimental.pallas import tpu as pltpu
from jax.experimental.pallas import tpu_sc as plsc

_N = 16384
_K = 8192
_D = 256
_TN = 256


_SEG_BOUNDS = ((0, 2736), (2736, 5472), (5472, _K))


def _argmin_body(x_ref, w_ref, a2_ref, b2_ref, out_ref):
    x = x_ref[...]
    mm = lax.dot_general(x, w_ref[...], (((1,), (1,)), ((), ())))
    d2 = a2_ref[...] + b2_ref[...] - 2.0 * mm
    dist = jnp.sqrt(jnp.maximum(d2, 0.0))
    kio = lax.broadcasted_iota(jnp.int32, dist.shape, 1)
    acc_v = None
    for a, b in _SEG_BOUNDS:
        mask = (kio >= a) & (kio < b)
        dseg = jnp.where(mask, dist, jnp.inf)
        v = jnp.min(dseg, axis=1, keepdims=True)
        i = jnp.min(jnp.where(dseg == v, kio, _K), axis=1, keepdims=True)
        vr = v.astype(jnp.bfloat16).astype(jnp.float32)
        if acc_v is None:
            acc_v, acc_i = vr, i
        else:
            upd = v < acc_v
            acc_v = jnp.where(upd, vr, acc_v)
            acc_i = jnp.where(upd, i, acc_i)
    out_ref[...] = acc_i[:, 0].astype(jnp.int32)


def _argmin_tc(flat, W, a2, b2):
    grid = (_N // _TN,)
    return pl.pallas_call(
        _argmin_body,
        grid=grid,
        in_specs=[
            pl.BlockSpec((_TN, _D), lambda i: (i, 0)),
            pl.BlockSpec((_K, _D), lambda i: (0, 0)),
            pl.BlockSpec((_TN, 1), lambda i: (i, 0)),
            pl.BlockSpec((1, _K), lambda i: (0, 0)),
        ],
        out_specs=pl.BlockSpec((_TN,), lambda i: (i,)),
        out_shape=jax.ShapeDtypeStruct((_N,), jnp.int32),
    )(flat, W, a2, b2)


def _gather_sc(W, idx):
    info = plsc.get_sparse_core_info()
    nw = info.num_cores * info.num_subcores
    per_w = _N // nw
    ch = 128
    n_ch = per_w // ch
    mesh = plsc.VectorSubcoreMesh(core_axis_name="c", subcore_axis_name="s")

    @functools.partial(
        pl.kernel,
        mesh=mesh,
        out_type=jax.ShapeDtypeStruct((_N, _D), jnp.float32),
        scratch_types=[
            pltpu.VMEM((ch,), jnp.int32),
            pltpu.VMEM((ch, _D), jnp.float32),
            pltpu.SemaphoreType.DMA,
        ],
    )
    def k(table_hbm, idx_hbm, out_hbm, idx_v, rows_v, sem):
        wid = lax.axis_index("s") * info.num_cores + lax.axis_index("c")
        base = wid * per_w
        for i in range(n_ch):
            off = base + i * ch
            pltpu.sync_copy(idx_hbm.at[pl.ds(off, ch)], idx_v)
            pltpu.async_copy(table_hbm.at[idx_v], rows_v, sem).wait()
            pltpu.sync_copy(rows_v, out_hbm.at[pl.ds(off, ch)])

    return k(W, idx)


def kernel(inp, W):
    flat = inp.reshape(-1, _D)
    a2 = jnp.sum(flat * flat, axis=1, keepdims=True)
    b2 = jnp.sum(W * W, axis=1)[None, :]
    idx = _argmin_tc(flat, W, a2, b2)
    rows = _gather_sc(W, idx)
    return rows.reshape(inp.shape)


# speedup vs baseline: 1.2062x; 1.2062x over previous
import functools

import jax
import jax.numpy as jnp
from jax import lax
from jax.experimental import pallas as pl
from jax.experimental.pallas import tpu as pltpu
from jax.experimental.pallas import tpu_sc as plsc

_N = 16384
_K = 8192
_D = 256
_TN = 256


_SEG_BOUNDS = ((0, 2736), (2736, 5472), (5472, _K))


def _seg_cols(a, b):
    cols = []
    for c in range(a // 128, (b + 127) // 128):
        lo = max(a - c * 128, 0)
        hi = min(b - c * 128, 128)
        cols.append((c, lo, hi))
    return cols


def _argmin_body(x_ref, w_ref, a2_ref, b2_ref, out_ref):
    x = x_ref[...]
    mm = lax.dot_general(x, w_ref[...], (((1,), (1,)), ((), ())))
    d2 = a2_ref[...] + b2_ref[...] - 2.0 * mm
    dist = jnp.sqrt(jnp.maximum(d2, 0.0))
    lio = lax.broadcasted_iota(jnp.int32, (_TN, 128), 1)
    acc_v = None
    for a, b in _SEG_BOUNDS:
        cols = _seg_cols(a, b)
        seg = None
        for c, lo, hi in cols:
            d = dist[:, c * 128:(c + 1) * 128]
            if (lo, hi) != (0, 128):
                d = jnp.where((lio >= lo) & (lio < hi), d, jnp.inf)
            seg = d if seg is None else jnp.minimum(seg, d)
        v = jnp.min(seg, axis=1, keepdims=True)
        iacc = None
        for c, lo, hi in cols:
            d = dist[:, c * 128:(c + 1) * 128]
            cand = jnp.where(d == v, lio + c * 128, _K)
            if (lo, hi) != (0, 128):
                cand = jnp.where((lio >= lo) & (lio < hi), cand, _K)
            iacc = cand if iacc is None else jnp.minimum(iacc, cand)
        i = jnp.min(iacc, axis=1, keepdims=True)
        vr = v.astype(jnp.bfloat16).astype(jnp.float32)
        if acc_v is None:
            acc_v, acc_i = vr, i
        else:
            upd = v < acc_v
            acc_v = jnp.where(upd, vr, acc_v)
            acc_i = jnp.where(upd, i, acc_i)
    out_ref[...] = acc_i[:, 0].astype(jnp.int32)


def _argmin_tc(flat, W, a2, b2):
    grid = (_N // _TN,)
    return pl.pallas_call(
        _argmin_body,
        grid=grid,
        in_specs=[
            pl.BlockSpec((_TN, _D), lambda i: (i, 0)),
            pl.BlockSpec((_K, _D), lambda i: (0, 0)),
            pl.BlockSpec((_TN, 1), lambda i: (i, 0)),
            pl.BlockSpec((1, _K), lambda i: (0, 0)),
        ],
        out_specs=pl.BlockSpec((_TN,), lambda i: (i,)),
        out_shape=jax.ShapeDtypeStruct((_N,), jnp.int32),
    )(flat, W, a2, b2)


def _gather_sc(W, idx):
    info = plsc.get_sparse_core_info()
    nw = info.num_cores * info.num_subcores
    per_w = _N // nw
    ch = 128
    n_ch = per_w // ch
    mesh = plsc.VectorSubcoreMesh(core_axis_name="c", subcore_axis_name="s")

    @functools.partial(
        pl.kernel,
        mesh=mesh,
        out_type=jax.ShapeDtypeStruct((_N, _D), jnp.float32),
        scratch_types=[
            pltpu.VMEM((ch,), jnp.int32),
            pltpu.VMEM((ch, _D), jnp.float32),
            pltpu.SemaphoreType.DMA,
        ],
    )
    def k(table_hbm, idx_hbm, out_hbm, idx_v, rows_v, sem):
        wid = lax.axis_index("s") * info.num_cores + lax.axis_index("c")
        base = wid * per_w
        for i in range(n_ch):
            off = base + i * ch
            pltpu.sync_copy(idx_hbm.at[pl.ds(off, ch)], idx_v)
            pltpu.async_copy(table_hbm.at[idx_v], rows_v, sem).wait()
            pltpu.sync_copy(rows_v, out_hbm.at[pl.ds(off, ch)])

    return k(W, idx)


def kernel(inp, W):
    flat = inp.reshape(-1, _D)
    a2 = jnp.sum(flat * flat, axis=1, keepdims=True)
    b2 = jnp.sum(W * W, axis=1)[None, :]
    idx = _argmin_tc(flat, W, a2, b2)
    rows = _gather_sc(W, idx)
    return rows.reshape(inp.shape)


# speedup vs baseline: 1.2862x; 1.0663x over previous
import functools

import jax
import jax.numpy as jnp
from jax import lax
from jax.experimental import pallas as pl
from jax.experimental.pallas import tpu as pltpu
from jax.experimental.pallas import tpu_sc as plsc

_N = 16384
_K = 8192
_D = 256
_TN = 512


_SEG_BOUNDS = ((0, 2736), (2736, 5472), (5472, _K))


def _seg_cols(a, b):
    cols = []
    for c in range(a // 128, (b + 127) // 128):
        lo = max(a - c * 128, 0)
        hi = min(b - c * 128, 128)
        cols.append((c, lo, hi))
    return cols


def _argmin_body(x_ref, w_ref, a2_ref, b2_ref, out_ref):
    x = x_ref[...]
    mm = lax.dot_general(x, w_ref[...], (((1,), (1,)), ((), ())))
    d2 = a2_ref[...] + b2_ref[...] - 2.0 * mm
    dist = jnp.sqrt(jnp.maximum(d2, 0.0))
    lio = lax.broadcasted_iota(jnp.int32, (_TN, 128), 1)
    acc_v = None
    for a, b in _SEG_BOUNDS:
        cols = _seg_cols(a, b)
        seg = None
        for c, lo, hi in cols:
            d = dist[:, c * 128:(c + 1) * 128]
            if (lo, hi) != (0, 128):
                d = jnp.where((lio >= lo) & (lio < hi), d, jnp.inf)
            seg = d if seg is None else jnp.minimum(seg, d)
        v = jnp.min(seg, axis=1, keepdims=True)
        iacc = None
        for c, lo, hi in cols:
            d = dist[:, c * 128:(c + 1) * 128]
            cand = jnp.where(d == v, lio + c * 128, _K)
            if (lo, hi) != (0, 128):
                cand = jnp.where((lio >= lo) & (lio < hi), cand, _K)
            iacc = cand if iacc is None else jnp.minimum(iacc, cand)
        i = jnp.min(iacc, axis=1, keepdims=True)
        vr = v.astype(jnp.bfloat16).astype(jnp.float32)
        if acc_v is None:
            acc_v, acc_i = vr, i
        else:
            upd = v < acc_v
            acc_v = jnp.where(upd, vr, acc_v)
            acc_i = jnp.where(upd, i, acc_i)
    out_ref[...] = acc_i[:, 0].astype(jnp.int32)


def _argmin_tc(flat, W, a2, b2):
    grid = (_N // _TN,)
    return pl.pallas_call(
        _argmin_body,
        grid=grid,
        in_specs=[
            pl.BlockSpec((_TN, _D), lambda i: (i, 0)),
            pl.BlockSpec((_K, _D), lambda i: (0, 0)),
            pl.BlockSpec((_TN, 1), lambda i: (i, 0)),
            pl.BlockSpec((1, _K), lambda i: (0, 0)),
        ],
        out_specs=pl.BlockSpec((_TN,), lambda i: (i,)),
        out_shape=jax.ShapeDtypeStruct((_N,), jnp.int32),
    )(flat, W, a2, b2)


def _gather_sc(W, idx):
    info = plsc.get_sparse_core_info()
    nw = info.num_cores * info.num_subcores
    per_w = _N // nw
    ch = 128
    n_ch = per_w // ch
    mesh = plsc.VectorSubcoreMesh(core_axis_name="c", subcore_axis_name="s")

    @functools.partial(
        pl.kernel,
        mesh=mesh,
        out_type=jax.ShapeDtypeStruct((_N, _D), jnp.float32),
        scratch_types=[
            pltpu.VMEM((ch,), jnp.int32),
            pltpu.VMEM((ch, _D), jnp.float32),
            pltpu.SemaphoreType.DMA,
        ],
    )
    def k(table_hbm, idx_hbm, out_hbm, idx_v, rows_v, sem):
        wid = lax.axis_index("s") * info.num_cores + lax.axis_index("c")
        base = wid * per_w
        for i in range(n_ch):
            off = base + i * ch
            pltpu.sync_copy(idx_hbm.at[pl.ds(off, ch)], idx_v)
            pltpu.async_copy(table_hbm.at[idx_v], rows_v, sem).wait()
            pltpu.sync_copy(rows_v, out_hbm.at[pl.ds(off, ch)])

    return k(W, idx)


def kernel(inp, W):
    flat = inp.reshape(-1, _D)
    a2 = jnp.sum(flat * flat, axis=1, keepdims=True)
    b2 = jnp.sum(W * W, axis=1)[None, :]
    idx = _argmin_tc(flat, W, a2, b2)
    rows = _gather_sc(W, idx)
    return rows.reshape(inp.shape)
